# baseline (speedup 1.0000x reference)
import functools

import jax
import jax.numpy as jnp
from jax import lax
from jax.experimental import pallas as pl
from jax.experimental.pallas import tpu as pltpu
from jax.experimental.pallas import tpu_sc as plsc

BATCH = 16384
EMB_K = 16

_NC, _NS = 2, 16
_BPT = BATCH // _NS
_CHB = 128
_NCH = _BPT // _CHB
_L = 16
_TW = 100096


@functools.cache
def _make_sc_gather():
    mesh = plsc.VectorSubcoreMesh(core_axis_name="c", subcore_axis_name="s")

    @functools.partial(
        pl.kernel,
        mesh=mesh,
        compiler_params=pltpu.CompilerParams(use_tc_tiling_on_sc=False,
                                             needs_layout_passes=False),
        out_type=[
            jax.ShapeDtypeStruct((EMB_K, BATCH), jnp.float32),
            jax.ShapeDtypeStruct((EMB_K, BATCH), jnp.float32),
            jax.ShapeDtypeStruct((BATCH,), jnp.float32),
            jax.ShapeDtypeStruct((BATCH,), jnp.float32),
        ],
        scratch_types=[
            pltpu.VMEM_SHARED((EMB_K * _TW,), jnp.float32),
            pltpu.VMEM((_BPT,), jnp.int32),
            pltpu.VMEM((EMB_K, _CHB), jnp.int32),
            pltpu.VMEM((EMB_K, _BPT), jnp.float32),
            pltpu.VMEM((_BPT,), jnp.float32),
            pltpu.SemaphoreType.DMA,
        ],
    )
    def gather_kernel(xt_hbm, tabs_hbm, ub_hbm, ib_hbm,
                      uzt_out, vzt_out, ubg_out, ibg_out,
                      spm, idx_v, idxk_v, zt_v, br_v, sem):
        cid = lax.axis_index("c")
        sid = lax.axis_index("s")

        def side(trow_base, bias_hbm, xoff, zt_out, bg_out):
            pltpu.sync_copy(tabs_hbm.at[sid + trow_base],
                            spm.at[pl.ds(sid * _TW, _TW)])
            plsc.subcore_barrier()

            gbase = sid * _BPT
            pltpu.async_copy(
                xt_hbm.at[pl.ds(xoff + gbase, _BPT)], idx_v, sem).wait()

            def chunk(j, carry):
                r = pl.ds(j * _CHB, _CHB)
                bias_cp = pltpu.async_copy(
                    bias_hbm.at[idx_v.at[r]], br_v.at[r], sem)
                for k in range(EMB_K):
                    for g in range(_CHB // _L):
                        s = pl.ds(g * _L, _L)
                        idxk_v[k, s] = idx_v[pl.ds(j * _CHB + g * _L, _L)] \
                            + (k * _TW)
                cps = [pltpu.async_copy(
                    spm.at[idxk_v.at[k]],
                    zt_v.at[k, pl.ds(j * _CHB, _CHB)], sem)
                    for k in range(EMB_K)]
                for cp in cps:
                    cp.wait()
                bias_cp.wait()
                return carry

            lax.fori_loop(0, _NCH, chunk, 0)

            out_cps = [
                pltpu.async_copy(zt_v, zt_out.at[:, pl.ds(gbase, _BPT)], sem),
                pltpu.async_copy(br_v, bg_out.at[pl.ds(gbase, _BPT)], sem),
            ]
            for cp in out_cps:
                cp.wait()

        @pl.when(cid == 0)
        def _():
            side(0, ub_hbm, 0, uzt_out, ubg_out)

        @pl.when(cid == 1)
        def _():
            side(EMB_K, ib_hbm, BATCH, vzt_out, ibg_out)

    return gather_kernel


_BLK = 4096


def _mlp_body(uzt_ref, vzt_ref, ub_ref, ib_ref, w1_ref, b1_ref, w2_ref,
              out_ref):
    uzt = uzt_ref[...]
    vzt = vzt_ref[...]
    w1 = w1_ref[...]
    h = lax.dot_general(w1[:, :EMB_K], uzt, (((1,), (0,)), ((), ())),
                        preferred_element_type=jnp.float32)
    h = h + lax.dot_general(w1[:, EMB_K:], vzt, (((1,), (0,)), ((), ())),
                            preferred_element_type=jnp.float32)
    h = jnp.maximum(h + b1_ref[...], 0.0)
    out = lax.dot_general(w2_ref[...], h, (((1,), (0,)), ((), ())),
                          preferred_element_type=jnp.float32)
    out_ref[...] = out + ub_ref[...] + ib_ref[...]


def _mlp(uzt, vzt, ub, ib, w1, b1, w2):
    grid = (BATCH // _BLK,)
    col_blk = lambda i: (0, i)
    w_blk = lambda i: (0, 0)
    return pl.pallas_call(
        _mlp_body,
        grid=grid,
        in_specs=[
            pl.BlockSpec((EMB_K, _BLK), col_blk),
            pl.BlockSpec((EMB_K, _BLK), col_blk),
            pl.BlockSpec((1, _BLK), col_blk),
            pl.BlockSpec((1, _BLK), col_blk),
            pl.BlockSpec((EMB_K, 2 * EMB_K), w_blk),
            pl.BlockSpec((EMB_K, 1), w_blk),
            pl.BlockSpec((1, EMB_K), w_blk),
        ],
        out_specs=pl.BlockSpec((1, _BLK), col_blk),
        out_shape=jax.ShapeDtypeStruct((1, BATCH), jnp.float32),
    )(uzt, vzt, ub, ib, w1, b1, w2)


def kernel(x, W, H, lin1_w, lin1_b, lin2_w, user_bias, item_bias):
    xt = x.T.reshape(-1)
    tabs = jnp.pad(jnp.concatenate([W.T, H.T], axis=0),
                   ((0, 0), (0, _TW - W.shape[0])))
    ubf = user_bias.T.reshape(-1)
    ibf = item_bias.T.reshape(-1)
    uzt, vzt, ubg, ibg = _make_sc_gather()(xt, tabs, ubf, ibf)
    out = _mlp(uzt, vzt, ubg.reshape(1, BATCH), ibg.reshape(1, BATCH),
               lin1_w, lin1_b.reshape(EMB_K, 1), lin2_w)
    return out.reshape(BATCH, 1)


# speedup vs baseline: 1.0114x; 1.0114x over previous
import functools

import jax
import jax.numpy as jnp
from jax import lax
from jax.experimental import pallas as pl
from jax.experimental.pallas import tpu as pltpu
from jax.experimental.pallas import tpu_sc as plsc

BATCH = 16384
EMB_K = 16

_NC, _NS = 2, 16
_BPT = BATCH // _NS
_CHB = 128
_NCH = _BPT // _CHB
_L = 16
_TW = 100096


@functools.cache
def _make_sc_gather():
    mesh = plsc.VectorSubcoreMesh(core_axis_name="c", subcore_axis_name="s")

    @functools.partial(
        pl.kernel,
        mesh=mesh,
        compiler_params=pltpu.CompilerParams(use_tc_tiling_on_sc=False,
                                             needs_layout_passes=False),
        out_type=[
            jax.ShapeDtypeStruct((EMB_K, BATCH), jnp.float32),
            jax.ShapeDtypeStruct((EMB_K, BATCH), jnp.float32),
            jax.ShapeDtypeStruct((BATCH,), jnp.float32),
            jax.ShapeDtypeStruct((BATCH,), jnp.float32),
        ],
        scratch_types=[
            pltpu.VMEM_SHARED((EMB_K * _TW,), jnp.float32),
            pltpu.VMEM((_BPT,), jnp.int32),
            pltpu.VMEM((EMB_K, _CHB), jnp.int32),
            pltpu.VMEM((EMB_K, _BPT), jnp.float32),
            pltpu.VMEM((_BPT,), jnp.float32),
            pltpu.SemaphoreType.DMA,
        ],
    )
    def gather_kernel(xt_hbm, tabs_hbm, ub_hbm, ib_hbm,
                      uzt_out, vzt_out, ubg_out, ibg_out,
                      spm, idx_v, idxk_v, zt_v, br_v, sem):
        cid = lax.axis_index("c")
        sid = lax.axis_index("s")

        def side(trow_base, bias_hbm, xoff, zt_out, bg_out):
            pltpu.sync_copy(tabs_hbm.at[sid + trow_base],
                            spm.at[pl.ds(sid * _TW, _TW)])
            plsc.subcore_barrier()

            gbase = sid * _BPT
            pltpu.async_copy(
                xt_hbm.at[pl.ds(xoff + gbase, _BPT)], idx_v, sem).wait()

            def chunk(j, carry):
                r = pl.ds(j * _CHB, _CHB)
                bias_cp = pltpu.async_copy(
                    bias_hbm.at[idx_v.at[r]], br_v.at[r], sem)
                for k in range(EMB_K):
                    for g in range(_CHB // _L):
                        s = pl.ds(g * _L, _L)
                        idxk_v[k, s] = idx_v[pl.ds(j * _CHB + g * _L, _L)] \
                            + (k * _TW)
                cps = [pltpu.async_copy(
                    spm.at[idxk_v.at[k]],
                    zt_v.at[k, pl.ds(j * _CHB, _CHB)], sem)
                    for k in range(EMB_K)]
                for cp in cps:
                    cp.wait()
                bias_cp.wait()
                return carry

            lax.fori_loop(0, _NCH, chunk, 0)

            out_cps = [
                pltpu.async_copy(zt_v, zt_out.at[:, pl.ds(gbase, _BPT)], sem),
                pltpu.async_copy(br_v, bg_out.at[pl.ds(gbase, _BPT)], sem),
            ]
            for cp in out_cps:
                cp.wait()

        @pl.when(cid == 0)
        def _():
            side(0, ub_hbm, 0, uzt_out, ubg_out)

        @pl.when(cid == 1)
        def _():
            side(EMB_K, ib_hbm, BATCH, vzt_out, ibg_out)

    return gather_kernel


_BLK = 4096


def _mlp_body(uzt_ref, vzt_ref, ub_ref, ib_ref, w1_ref, b1_ref, w2_ref,
              out_ref):
    uzt = uzt_ref[...]
    vzt = vzt_ref[...]
    w1 = w1_ref[...]
    h = lax.dot_general(w1[:, :EMB_K], uzt, (((1,), (0,)), ((), ())),
                        preferred_element_type=jnp.float32)
    h = h + lax.dot_general(w1[:, EMB_K:], vzt, (((1,), (0,)), ((), ())),
                            preferred_element_type=jnp.float32)
    h = jnp.maximum(h + b1_ref[...], 0.0)
    out = lax.dot_general(w2_ref[...], h, (((1,), (0,)), ((), ())),
                          preferred_element_type=jnp.float32)
    out_ref[...] = out + ub_ref[...] + ib_ref[...]


def _mlp(uzt, vzt, ub, ib, w1, b1, w2):
    grid = (BATCH // _BLK,)
    col_blk = lambda i: (0, i)
    w_blk = lambda i: (0, 0)
    return pl.pallas_call(
        _mlp_body,
        grid=grid,
        in_specs=[
            pl.BlockSpec((EMB_K, _BLK), col_blk),
            pl.BlockSpec((EMB_K, _BLK), col_blk),
            pl.BlockSpec((1, _BLK), col_blk),
            pl.BlockSpec((1, _BLK), col_blk),
            pl.BlockSpec((EMB_K, 2 * EMB_K), w_blk),
            pl.BlockSpec((EMB_K, 1), w_blk),
            pl.BlockSpec((1, EMB_K), w_blk),
        ],
        out_specs=pl.BlockSpec((1, _BLK), col_blk),
        out_shape=jax.ShapeDtypeStruct((1, BATCH), jnp.float32),
    )(uzt, vzt, ub, ib, w1, b1, w2)


def kernel(x, W, H, lin1_w, lin1_b, lin2_w, user_bias, item_bias):
    xt = x.T.reshape(-1)
    pad = ((0, 0), (0, _TW - W.shape[0]))
    tabs = jnp.concatenate([jnp.pad(W.T, pad), jnp.pad(H.T, pad)], axis=0)
    ubf = user_bias.T.reshape(-1)
    ibf = item_bias.T.reshape(-1)
    uzt, vzt, ubg, ibg = _make_sc_gather()(xt, tabs, ubf, ibf)
    out = _mlp(uzt, vzt, ubg.reshape(1, BATCH), ibg.reshape(1, BATCH),
               lin1_w, lin1_b.reshape(EMB_K, 1), lin2_w)
    return out.reshape(BATCH, 1)
